# Initial kernel scaffold; baseline (speedup 1.0000x reference)
import jax
import jax.numpy as jnp
from jax.experimental import pallas as pl


def kernel(x, edge_index, edge_weight, W_z, b_z, lz_W, lz_b, W_r, b_r, lr_W, lr_b, W_h, b_h, lh_W, lh_b, att, p1_W, p1_b, p2_W, p2_b):
    raise NotImplementedError("write your pallas kernel here")


# speedup vs baseline: 15.2448x; 15.2448x over previous
import functools

import jax
import jax.numpy as jnp
from jax import lax
from jax.experimental import pallas as pl
from jax.experimental.pallas import tpu as pltpu

N = 50000
F_IN = 60
HID = 64
P = 12
BN = 400


def _dense_body(yr_ref, seg_ref, Wz_ref, Lz_ref, Wh_ref, Lh_ref, czh_ref,
                p1W_ref, p1b_ref, p2W_ref, p2b_ref, out_ref):
    Mz = Wz_ref[...] @ Lz_ref[...]
    Mh = Wh_ref[...] @ Lh_ref[...]
    A = yr_ref[...] @ jnp.concatenate([Mz, Mh], axis=1) + czh_ref[...]
    Z = jax.nn.sigmoid(A[:, :HID])
    T = jnp.tanh(A[:, HID:])
    G = (1.0 - Z) * T
    H = lax.dot_general(seg_ref[...], G, (((0,), (0,)), ((), ())))
    h = jax.nn.relu(H @ p1W_ref[...] + p1b_ref[...])
    out_ref[...] = h @ p2W_ref[...] + p2b_ref[...]


def _dense_stage(Yr, seg, Wz, Lz, Wh, Lh, czh, p1W, p1b, p2W, p2b):
    nb = N // BN
    full = lambda shape: pl.BlockSpec(shape, lambda i: (0,) * len(shape))
    return pl.pallas_call(
        _dense_body,
        grid=(nb,),
        in_specs=[
            pl.BlockSpec((BN * P, F_IN), lambda i: (i, 0)),
            full((BN * P, BN)),
            full((F_IN, HID)), full((HID, HID)),
            full((F_IN, HID)), full((HID, HID)),
            full((1, 2 * HID)),
            full((HID, HID)), full((1, HID)),
            full((HID, 1)), full((1, 1)),
        ],
        out_specs=pl.BlockSpec((BN, 1), lambda i: (i, 0)),
        out_shape=jax.ShapeDtypeStruct((N, 1), jnp.float32),
    )(Yr, seg, Wz, Lz, Wh, Lh, czh, p1W, p1b, p2W, p2b)


def kernel(x, edge_index, edge_weight, W_z, b_z, lz_W, lz_b, W_r, b_r, lr_W,
           lr_b, W_h, b_h, lh_W, lh_b, att, p1_W, p1_b, p2_W, p2_b):
    src, dst = edge_index[0], edge_index[1]
    n = x.shape[0]

    deg = jnp.ones((n,), jnp.float32).at[dst].add(edge_weight)
    dinv = lax.rsqrt(deg)
    Xt = x.transpose(0, 2, 1).reshape(n, P * F_IN)
    Xs = Xt * dinv[:, None]
    Y0 = jnp.zeros((n, P * F_IN), jnp.float32).at[dst].add(
        Xs[src] * edge_weight[:, None])
    Y = (Y0 + Xs) * dinv[:, None]

    Lz = lz_W[:HID]
    Lh = lh_W[:HID]
    czh = jnp.concatenate([b_z @ Lz + lz_b, b_h @ Lh + lh_b])[None, :]
    probs = jax.nn.softmax(att)
    j = jnp.arange(BN * P)
    seg = jnp.where(j[:, None] // P == jnp.arange(BN)[None, :],
                    probs[j % P][:, None], 0.0).astype(jnp.float32)
    Yr = Y.reshape(n * P, F_IN)
    return _dense_stage(Yr, seg, W_z, Lz, W_h, Lh, czh,
                        p1_W, p1_b[None, :], p2_W, p2_b[None, :])
